# Initial kernel scaffold; baseline (speedup 1.0000x reference)
import jax
import jax.numpy as jnp
from jax.experimental import pallas as pl


def kernel(qf0, qf1, qf2, qf3, sf0, sf1, sf2, sf3, vf0, vf1, vf2, vf3, q1_dw, q1_pw, q1_b, q2_dw, q2_pw, q2_b, q3_dw, q3_pw, q3_b, k1_dw, k1_pw, k1_b, k2_dw, k2_pw, k2_b, k3_dw, k3_pw, k3_b):
    raise NotImplementedError("write your pallas kernel here")


# speedup vs baseline: 1.0097x; 1.0097x over previous
import functools

import jax
import jax.numpy as jnp
from jax.experimental import pallas as pl
from jax.experimental.pallas import tpu as pltpu

_THRESH = 0.95
_NEG = -1000000000.0


def _b16(x):
    return x.astype(jnp.bfloat16).astype(jnp.float32)


def _dotb(a, b, dims):
    return jax.lax.dot_general(a.astype(jnp.bfloat16), b.astype(jnp.bfloat16),
                               (dims, ((), ())),
                               preferred_element_type=jnp.float32)


def _doth(a, b, dims):
    return jax.lax.dot_general(a, b, (dims, ((), ())),
                               preferred_element_type=jnp.float32,
                               precision=jax.lax.Precision.HIGHEST)


def _sda_kernel(supp_ref, qry_ref, vtok_ref,
                qdw_ref, qpw_ref, qb_ref, kdw_ref, kpw_ref, kb_ref,
                out_ref, qtok_ref, ktok_ref, selq_ref, selk_ref,
                *, H, W, hr, wr, C, qchunk, pw_b16w, v_b16):
    sh, sw = H // hr, W // wr
    BS = sh * sw
    NB = hr * wr
    N = BS * NB
    t = pl.program_id(1)

    def dwconv(img, w3):
        zh = jnp.zeros((1, W, C), jnp.float32)
        xp = jnp.concatenate([zh, img, zh], axis=0)
        zw = jnp.zeros((H + 2, 1, C), jnp.float32)
        xp = jnp.concatenate([zw, xp, zw], axis=1)
        acc = None
        for kh in range(3):
            for kw in range(3):
                term = xp[kh:kh + H, kw:kw + W, :] * w3[kh, kw][None, None, :]
                acc = term if acc is None else acc + term
        return acc

    def conv(img, dw, pw, bias):
        x = dwconv(img, dw) / jnp.sqrt(jnp.float32(1.0 + 1e-05))
        xt = x.reshape(H * W, C)
        if pw_b16w:
            y = _dotb(xt, pw, ((1,), (0,)))
        else:
            y = _doth(xt, pw, ((1,), (0,)))
        return y + bias

    def write_blocks(y, ref):
        y3 = y.reshape(H, W, C)
        for bi in range(NB):
            j, i = divmod(bi, wr)
            blk = y3[j * sh:(j + 1) * sh, i * sw:(i + 1) * sw, :]
            ref[bi * BS:(bi + 1) * BS, :] = blk.reshape(BS, C)

    @pl.when(t == 0)
    def _():
        q = conv(supp_ref[0], qdw_ref[...], qpw_ref[...], qb_ref[...])
        write_blocks(q, qtok_ref)

    k = conv(qry_ref[0, 0], kdw_ref[...], kpw_ref[...], kb_ref[...])
    write_blocks(k, ktok_ref)

    for bi in range(NB):
        r0 = bi * BS
        qb = qtok_ref[r0:r0 + BS, :]
        kb = ktok_ref[r0:r0 + BS, :]
        qn = qb / jnp.maximum(jnp.sqrt(jnp.sum(qb * qb, axis=1, keepdims=True)), 1e-12)
        kn = kb / jnp.maximum(jnp.sqrt(jnp.sum(kb * kb, axis=1, keepdims=True)), 1e-12)
        sim = _doth(qn, kn, ((1,), (1,)))
        hit = (sim >= _THRESH).astype(jnp.float32)
        selq_ref[r0:r0 + BS, :] = (jnp.sum(hit, axis=1, keepdims=True) == 0.0).astype(jnp.float32)
        selk_ref[:, r0:r0 + BS] = (jnp.sum(hit, axis=0, keepdims=True) == 0.0).astype(jnp.float32)
        m = jnp.max(sim, axis=1, keepdims=True)
        e = jnp.exp(sim - m)
        attn = e / jnp.sum(e, axis=1, keepdims=True)
        vb = vtok_ref[0, 0, r0:r0 + BS, :]
        if v_b16:
            vb = _b16(vb)
        out_ref[0, 0, r0:r0 + BS, :] = _doth(attn, vb, ((1,), (0,)))

    kall = ktok_ref[:, :]
    vall = vtok_ref[0, 0, :, :]
    selk = selk_ref[:, :]
    for ci in range(N // qchunk):
        r0 = ci * qchunk
        qc = qtok_ref[r0:r0 + qchunk, :]
        s = _doth(qc, kall, ((1,), (1,)))
        s = jnp.where(selk > 0.0, s, _NEG)
        m = jnp.max(s, axis=1, keepdims=True)
        e = jnp.exp(s - m)
        og = _doth(e / jnp.sum(e, axis=1, keepdims=True), _b16(vall), ((1,), (0,)))
        sq = selq_ref[r0:r0 + qchunk, :] > 0.0
        out_ref[0, 0, r0:r0 + qchunk, :] = jnp.where(
            sq, og, out_ref[0, 0, r0:r0 + qchunk, :])


def _sda_level(supp, qry, val, dw_q, pw_q, b_q, dw_k, pw_k, b_k, hr, wr,
               qchunk, pw_b16w, v_b16):
    B, T, C, H, W = val.shape
    sh, sw = H // hr, W // wr
    N = H * W
    supp_i = supp[:, 0].transpose(0, 2, 3, 1)
    qry_i = qry.transpose(0, 1, 3, 4, 2)
    vtok = (val.reshape(B, T, C, hr, sh, wr, sw)
            .transpose(0, 1, 3, 5, 4, 6, 2).reshape(B, T, N, C))
    qdw = dw_q[:, 0].transpose(1, 2, 0)
    qpw = pw_q[:, :, 0, 0].T
    kdw = dw_k[:, 0].transpose(1, 2, 0)
    kpw = pw_k[:, :, 0, 0].T
    kern = functools.partial(_sda_kernel, H=H, W=W, hr=hr, wr=wr, C=C,
                             qchunk=qchunk, pw_b16w=pw_b16w, v_b16=v_b16)
    out = pl.pallas_call(
        kern,
        grid=(B, T),
        in_specs=[
            pl.BlockSpec((1, H, W, C), lambda b, t: (b, 0, 0, 0)),
            pl.BlockSpec((1, 1, H, W, C), lambda b, t: (b, t, 0, 0, 0)),
            pl.BlockSpec((1, 1, N, C), lambda b, t: (b, t, 0, 0)),
            pl.BlockSpec((3, 3, C), lambda b, t: (0, 0, 0)),
            pl.BlockSpec((C, C), lambda b, t: (0, 0)),
            pl.BlockSpec((1, C), lambda b, t: (0, 0)),
            pl.BlockSpec((3, 3, C), lambda b, t: (0, 0, 0)),
            pl.BlockSpec((C, C), lambda b, t: (0, 0)),
            pl.BlockSpec((1, C), lambda b, t: (0, 0)),
        ],
        out_specs=pl.BlockSpec((1, 1, N, C), lambda b, t: (b, t, 0, 0)),
        out_shape=jax.ShapeDtypeStruct((B, T, N, C), jnp.float32),
        scratch_shapes=[
            pltpu.VMEM((N, C), jnp.float32),
            pltpu.VMEM((N, C), jnp.float32),
            pltpu.VMEM((N, 1), jnp.float32),
            pltpu.VMEM((1, N), jnp.float32),
        ],
        compiler_params=pltpu.CompilerParams(
            dimension_semantics=("parallel", "arbitrary")),
    )(supp_i, qry_i, vtok, qdw, qpw, b_q[None, :], kdw, kpw, b_k[None, :])
    out = (out.reshape(B, T, hr, wr, sh, sw, C)
           .transpose(0, 1, 6, 2, 4, 3, 5).reshape(B, T, C, H, W))
    return out


def kernel(qf0, qf1, qf2, qf3, sf0, sf1, sf2, sf3, vf0, vf1, vf2, vf3,
           q1_dw, q1_pw, q1_b, q2_dw, q2_pw, q2_b, q3_dw, q3_pw, q3_b,
           k1_dw, k1_pw, k1_b, k2_dw, k2_pw, k2_b, k3_dw, k3_pw, k3_b):
    o0 = _sda_level(sf3, qf3, vf3, q3_dw, q3_pw, q3_b, k3_dw, k3_pw, k3_b,
                    1, 1, 144, False, True)
    o1 = _sda_level(sf2, qf2, vf2, q2_dw, q2_pw, q2_b, k2_dw, k2_pw, k2_b,
                    2, 2, 576, False, True)
    o2 = _sda_level(sf1, qf1, vf1, q1_dw, q1_pw, q1_b, k1_dw, k1_pw, k1_b,
                    4, 4, 576, False, False)
    return (o0, o1, o2)
